# baseline (speedup 1.0000x reference)
import jax
import jax.numpy as jnp
from jax.experimental import pallas as pl
from jax.experimental.pallas import tpu as pltpu


def _reg_body(g_ref, w1_ref, b1_ref, w2_ref, b2_ref, o_ref):
    r = jnp.maximum(g_ref[...] @ w1_ref[...] + b1_ref[...], 0.0)
    o_ref[...] = r @ w2_ref[...] + b2_ref[...]


def kernel(x, edge_index, pe, edge_attr, batch, W_phi, b_phi, W_rho, b_rho, W_edge, b_edge, Wq, bq, Wk, bk, Wv, bv, Wr1, br1, Wr2, br2):
    N, D = x.shape
    H = 8
    DH = D // H
    G = 256
    src = edge_index[0]
    dst = edge_index[1]
    h1 = jax.nn.relu(pe @ W_phi + b_phi) + jax.nn.relu((-pe) @ W_phi + b_phi)
    h = h1 @ W_rho + b_rho
    e = jax.nn.relu(edge_attr @ W_edge + b_edge)
    msg = h[src] * e
    m = jax.ops.segment_sum(msg, dst, num_segments=N)
    pos = h + m
    hn = x + pos
    q = (hn @ Wq + bq).reshape(N, H, DH)
    k = (hn @ Wk + bk).reshape(N, H, DH)
    v = (hn @ Wv + bv).reshape(N, H, DH)
    score = jnp.sum(q[dst] * k[src], axis=-1) / jnp.sqrt(float(DH))
    smax = jax.ops.segment_max(score, dst, num_segments=N)
    smax = jnp.where(jnp.isfinite(smax), smax, 0.0)
    ex = jnp.exp(score - jax.lax.stop_gradient(smax)[dst])
    denom = jax.ops.segment_sum(ex, dst, num_segments=N)
    alpha = ex / (denom[dst] + 1e-16)
    attn = jax.ops.segment_sum(alpha[..., None] * v[src], dst, num_segments=N)
    node = attn.reshape(-1, D)
    sums = jax.ops.segment_sum(node, batch, num_segments=G)
    cnt = jax.ops.segment_sum(jnp.ones((N, 1), dtype=jnp.float32), batch, num_segments=G)
    graph = sums / jnp.maximum(cnt, 1.0)
    out = pl.pallas_call(
        _reg_body,
        out_shape=jax.ShapeDtypeStruct((G, 1), jnp.float32),
    )(graph, Wr1, br1.reshape(1, -1), Wr2, br2.reshape(1, 1))
    return out


# speedup vs baseline: 1.0207x; 1.0207x over previous
import jax
import jax.numpy as jnp
from jax import lax
from jax.experimental import pallas as pl
from jax.experimental.pallas import tpu as pltpu
from jax.experimental.pallas import tpu_sc as plsc

_N = 10000
_E = 320000
_D = 128
_H = 8
_G = 256
_NC = 2
_NS = 16
_NW = _NC * _NS
_EP = 327680
_RPT = _EP // _NW // 128
_ACC_N = _N + 16
_NBLK = _ACC_N // 16
_OBLK = _N // 16


def _zero_acc(zbuf, acc_ref, s, nblk):
    def _zrow(i, _):
        zbuf[i // 8, pl.ds((i % 8) * 16, 16)] = jnp.zeros((16,), jnp.float32)
        return 0
    lax.fori_loop(0, 128, _zrow, 0)

    def _zcp(j, _):
        b = s + j * _NS
        @pl.when(b < nblk)
        def _():
            pltpu.sync_copy(zbuf.at[pl.ds(0, 16)],
                            acc_ref.at[pl.ds(pl.multiple_of(b * 16, 16), 16)])
        return 0
    lax.fori_loop(0, (nblk + _NS - 1) // _NS, _zcp, 0)


def _msg_sc_body(h_hbm, e_hbm, src_hbm, dst_hbm, out_hbm,
                 srcv, dstv, hrow, erow, acc_ref, sem):
    c = lax.axis_index("c")
    s = lax.axis_index("s")
    wid = c * _NS + s

    _zero_acc(erow, acc_ref, s, _NBLK)
    plsc.subcore_barrier()

    def _edge_chunk(lr, _):
        pltpu.sync_copy(src_hbm.at[wid, lr], srcv)
        pltpu.sync_copy(dst_hbm.at[wid, lr], dstv)
        pltpu.async_copy(h_hbm.at[srcv.at[0]], hrow, sem).wait()
        pltpu.sync_copy(e_hbm.at[wid * _RPT + lr], erow)

        def _mul(r, _):
            for k in range(8):
                sl = pl.ds(k * 16, 16)
                hrow[r, sl] = hrow[r, sl] * erow[r, sl]
            return 0
        lax.fori_loop(0, 128, _mul, 0)
        pltpu.sync_copy(hrow, acc_ref.at[dstv.at[0]], add=True)
        return 0
    lax.fori_loop(0, _RPT, _edge_chunk, 0)
    plsc.subcore_barrier()

    def _ocp(j, _):
        b = s + j * _NS
        @pl.when(b < _OBLK)
        def _():
            pltpu.sync_copy(acc_ref.at[pl.ds(pl.multiple_of(b * 16, 16), 16)],
                            out_hbm.at[c, b])
        return 0
    lax.fori_loop(0, (_OBLK + _NS - 1) // _NS, _ocp, 0)


def _msg_sc(h, e3, src4, dst4):
    mesh = plsc.VectorSubcoreMesh(core_axis_name="c", subcore_axis_name="s")
    f = pl.kernel(
        _msg_sc_body,
        out_type=jax.ShapeDtypeStruct((_NC, _OBLK, 16, _D), jnp.float32),
        mesh=mesh,
        scratch_types=[
            pltpu.VMEM((1, 128), jnp.int32),
            pltpu.VMEM((1, 128), jnp.int32),
            pltpu.VMEM((128, _D), jnp.float32),
            pltpu.VMEM((128, _D), jnp.float32),
            pltpu.VMEM_SHARED((_ACC_N, _D), jnp.float32),
            pltpu.SemaphoreType.DMA,
        ],
    )
    return f(h, e3, src4, dst4)


def kernel(x, edge_index, pe, edge_attr, batch, W_phi, b_phi, W_rho, b_rho, W_edge, b_edge, Wq, bq, Wk, bk, Wv, bv, Wr1, br1, Wr2, br2):
    N, D = x.shape
    H = _H
    DH = D // H
    G = _G
    src = edge_index[0]
    dst = edge_index[1]
    npad = _EP - _E
    src4 = jnp.concatenate([src, jnp.zeros((npad,), jnp.int32)]).reshape(_NW, _RPT, 1, 128)
    dst4 = jnp.concatenate([dst, jnp.full((npad,), _N, jnp.int32)]).reshape(_NW, _RPT, 1, 128)

    h1 = jax.nn.relu(pe @ W_phi + b_phi) + jax.nn.relu((-pe) @ W_phi + b_phi)
    h = h1 @ W_rho + b_rho
    ea_pad = jnp.concatenate([edge_attr, jnp.zeros((npad, edge_attr.shape[1]), jnp.float32)])
    e = jax.nn.relu(ea_pad @ W_edge + b_edge)
    e3 = e.reshape(_EP // 128, 128, _D)

    m2 = _msg_sc(h, e3, src4, dst4)
    m = (m2[0] + m2[1]).reshape(N, D)

    pos = h + m
    hn = x + pos
    q = (hn @ Wq + bq).reshape(N, H, DH)
    k = (hn @ Wk + bk).reshape(N, H, DH)
    v = (hn @ Wv + bv).reshape(N, H, DH)
    score = jnp.sum(q[dst] * k[src], axis=-1) / jnp.sqrt(float(DH))
    smax = jax.ops.segment_max(score, dst, num_segments=N)
    smax = jnp.where(jnp.isfinite(smax), smax, 0.0)
    ex = jnp.exp(score - jax.lax.stop_gradient(smax)[dst])
    denom = jax.ops.segment_sum(ex, dst, num_segments=N)
    alpha = ex / (denom[dst] + 1e-16)
    attn = jax.ops.segment_sum(alpha[..., None] * v[src], dst, num_segments=N)
    node = attn.reshape(-1, D)
    sums = jax.ops.segment_sum(node, batch, num_segments=G)
    cnt = jax.ops.segment_sum(jnp.ones((N, 1), dtype=jnp.float32), batch, num_segments=G)
    graph = sums / jnp.maximum(cnt, 1.0)
    r = jax.nn.relu(graph @ Wr1 + br1)
    out = r @ Wr2 + br2
    return out


# speedup vs baseline: 15.1304x; 14.8229x over previous
import jax
import jax.numpy as jnp
from jax import lax
from jax.experimental import pallas as pl
from jax.experimental.pallas import tpu as pltpu
from jax.experimental.pallas import tpu_sc as plsc

_N = 10000
_E = 320000
_D = 128
_H = 8
_G = 256
_NC = 2
_NS = 16
_NW = _NC * _NS
_EP = 327680
_RPT = _EP // _NW // 128
_ACC_N = _N + 16
_NBLK = _ACC_N // 16
_OBLK = _N // 16


def _zero_acc(zbuf, acc_ref, s, nblk):
    def _zrow(i, _):
        zbuf[i // 8, pl.ds((i % 8) * 16, 16)] = jnp.zeros((16,), jnp.float32)
        return 0
    lax.fori_loop(0, 128, _zrow, 0)

    def _zcp(j, _):
        b = s + j * _NS
        @pl.when(b < nblk)
        def _():
            pltpu.sync_copy(zbuf.at[pl.ds(0, 16)],
                            acc_ref.at[pl.ds(pl.multiple_of(b * 16, 16), 16)])
        return 0
    lax.fori_loop(0, (nblk + _NS - 1) // _NS, _zcp, 0)


def _mul_rows(arow, brow):
    def _mul(r, _):
        for k in range(8):
            sl = pl.ds(k * 16, 16)
            arow[r, sl] = arow[r, sl] * brow[r, sl]
        return 0
    lax.fori_loop(0, 128, _mul, 0)


def _msg_sc_body(h_hbm, e_hbm, src_hbm, dst_hbm, out_hbm,
                 srcv, dstv, hrow, erow, acc_ref, sem):
    c = lax.axis_index("c")
    s = lax.axis_index("s")
    wid = c * _NS + s

    _zero_acc(erow, acc_ref, s, _NBLK)
    plsc.subcore_barrier()

    def _edge_chunk(lr, _):
        pltpu.sync_copy(src_hbm.at[wid, lr], srcv)
        pltpu.sync_copy(dst_hbm.at[wid, lr], dstv)
        pltpu.async_copy(h_hbm.at[srcv.at[0]], hrow, sem).wait()
        pltpu.sync_copy(e_hbm.at[wid * _RPT + lr], erow)
        _mul_rows(hrow, erow)
        pltpu.sync_copy(hrow, acc_ref.at[dstv.at[0]], add=True)
        return 0
    lax.fori_loop(0, _RPT, _edge_chunk, 0)
    plsc.subcore_barrier()

    def _ocp(j, _):
        b = s + j * _NS
        @pl.when(b < _OBLK)
        def _():
            pltpu.sync_copy(acc_ref.at[pl.ds(pl.multiple_of(b * 16, 16), 16)],
                            out_hbm.at[c, b])
        return 0
    lax.fori_loop(0, (_OBLK + _NS - 1) // _NS, _ocp, 0)


def _msg_sc(h, e3, src4, dst4):
    mesh = plsc.VectorSubcoreMesh(core_axis_name="c", subcore_axis_name="s")
    f = pl.kernel(
        _msg_sc_body,
        out_type=jax.ShapeDtypeStruct((_NC, _OBLK, 16, _D), jnp.float32),
        mesh=mesh,
        scratch_types=[
            pltpu.VMEM((1, 128), jnp.int32),
            pltpu.VMEM((1, 128), jnp.int32),
            pltpu.VMEM((128, _D), jnp.float32),
            pltpu.VMEM((128, _D), jnp.float32),
            pltpu.VMEM_SHARED((_ACC_N, _D), jnp.float32),
            pltpu.SemaphoreType.DMA,
        ],
    )
    return f(h, e3, src4, dst4)


def _qk_sc_body(q_hbm, k_hbm, src_hbm, dst_hbm, out_hbm,
                srcv, dstv, qrow, krow, sem):
    c = lax.axis_index("c")
    s = lax.axis_index("s")
    wid = c * _NS + s

    def _edge_chunk(lr, _):
        pltpu.sync_copy(src_hbm.at[wid, lr], srcv)
        pltpu.sync_copy(dst_hbm.at[wid, lr], dstv)
        cp1 = pltpu.async_copy(q_hbm.at[dstv.at[0]], qrow, sem)
        cp2 = pltpu.async_copy(k_hbm.at[srcv.at[0]], krow, sem)
        cp1.wait()
        cp2.wait()
        _mul_rows(qrow, krow)
        pltpu.sync_copy(qrow, out_hbm.at[wid * _RPT + lr])
        return 0
    lax.fori_loop(0, _RPT, _edge_chunk, 0)


def _qk_sc(qp, kp, src4, dst4):
    mesh = plsc.VectorSubcoreMesh(core_axis_name="c", subcore_axis_name="s")
    f = pl.kernel(
        _qk_sc_body,
        out_type=jax.ShapeDtypeStruct((_EP // 128, 128, _D), jnp.float32),
        mesh=mesh,
        scratch_types=[
            pltpu.VMEM((1, 128), jnp.int32),
            pltpu.VMEM((1, 128), jnp.int32),
            pltpu.VMEM((128, _D), jnp.float32),
            pltpu.VMEM((128, _D), jnp.float32),
            pltpu.SemaphoreType.DMA,
        ],
    )
    return f(qp, kp, src4, dst4)


def _den_sc_body(ex_hbm, dst_hbm, out_hbm, dstv, xrow, acc_ref, sem):
    c = lax.axis_index("c")
    s = lax.axis_index("s")
    wid = c * _NS + s

    _zero_acc(xrow, acc_ref, s, _NBLK)
    plsc.subcore_barrier()

    def _edge_chunk(lr, _):
        pltpu.sync_copy(dst_hbm.at[wid, lr], dstv)
        pltpu.sync_copy(ex_hbm.at[wid * _RPT + lr], xrow)
        pltpu.sync_copy(xrow, acc_ref.at[dstv.at[0]], add=True)
        return 0
    lax.fori_loop(0, _RPT, _edge_chunk, 0)
    plsc.subcore_barrier()

    def _ocp(j, _):
        b = s + j * _NS
        @pl.when(b < _OBLK)
        def _():
            pltpu.sync_copy(acc_ref.at[pl.ds(pl.multiple_of(b * 16, 16), 16)],
                            out_hbm.at[c, b])
        return 0
    lax.fori_loop(0, (_OBLK + _NS - 1) // _NS, _ocp, 0)


def _den_sc(ex3, dst4):
    mesh = plsc.VectorSubcoreMesh(core_axis_name="c", subcore_axis_name="s")
    f = pl.kernel(
        _den_sc_body,
        out_type=jax.ShapeDtypeStruct((_NC, _OBLK, 16, _D), jnp.float32),
        mesh=mesh,
        scratch_types=[
            pltpu.VMEM((1, 128), jnp.int32),
            pltpu.VMEM((128, _D), jnp.float32),
            pltpu.VMEM_SHARED((_ACC_N, _D), jnp.float32),
            pltpu.SemaphoreType.DMA,
        ],
    )
    return f(ex3, dst4)


def kernel(x, edge_index, pe, edge_attr, batch, W_phi, b_phi, W_rho, b_rho, W_edge, b_edge, Wq, bq, Wk, bk, Wv, bv, Wr1, br1, Wr2, br2):
    N, D = x.shape
    H = _H
    DH = D // H
    G = _G
    src = edge_index[0]
    dst = edge_index[1]
    npad = _EP - _E
    src4 = jnp.concatenate([src, jnp.zeros((npad,), jnp.int32)]).reshape(_NW, _RPT, 1, 128)
    dst4 = jnp.concatenate([dst, jnp.full((npad,), _N, jnp.int32)]).reshape(_NW, _RPT, 1, 128)

    h1 = jax.nn.relu(pe @ W_phi + b_phi) + jax.nn.relu((-pe) @ W_phi + b_phi)
    h = h1 @ W_rho + b_rho
    ea_pad = jnp.concatenate([edge_attr, jnp.zeros((npad, edge_attr.shape[1]), jnp.float32)])
    e = jax.nn.relu(ea_pad @ W_edge + b_edge)
    e3 = e.reshape(_EP // 128, 128, _D)

    m2 = _msg_sc(h, e3, src4, dst4)
    m = (m2[0] + m2[1]).reshape(N, D)

    pos = h + m
    hn = x + pos
    q = hn @ Wq + bq
    k = hn @ Wk + bk
    v = hn @ Wv + bv
    zpad = jnp.zeros((_ACC_N - N, D), jnp.float32)
    qp = jnp.concatenate([q, zpad])
    kp = jnp.concatenate([k, zpad])
    vp = jnp.concatenate([v, zpad])

    prod3 = _qk_sc(qp, kp, src4, dst4)

    prod = prod3.reshape(_EP, _D)
    hid = jnp.arange(_D, dtype=jnp.int32) // DH
    red = (hid[:, None] == jnp.arange(H, dtype=jnp.int32)[None, :]).astype(jnp.float32)
    score = (prod @ red) * (1.0 / jnp.sqrt(float(DH)))
    ex = jnp.exp(score)
    expand = (jnp.arange(H, dtype=jnp.int32)[:, None] == hid[None, :]).astype(jnp.float32)
    exwide = ex @ expand
    ex3 = exwide.reshape(_EP // 128, 128, _D)

    attn2 = _msg_sc(vp, ex3, src4, dst4)
    attn = (attn2[0] + attn2[1]).reshape(N, D)
    den2 = _den_sc(ex3, dst4)
    denw = (den2[0] + den2[1]).reshape(N, D)
    node = attn / (denw + 1e-16)

    sums = jax.ops.segment_sum(node, batch, num_segments=G)
    cnt = jax.ops.segment_sum(jnp.ones((N, 1), dtype=jnp.float32), batch, num_segments=G)
    graph = sums / jnp.maximum(cnt, 1.0)
    r = jax.nn.relu(graph @ Wr1 + br1)
    out = r @ Wr2 + br2
    return out
